# baseline (speedup 1.0000x reference)
import numpy as np
import jax
import jax.numpy as jnp
from jax.experimental import pallas as pl
from jax.experimental.pallas import tpu as pltpu


def _anchor_table(box_size):
    ratios = np.asarray([0.5, 1.0, 2.0], dtype=np.float64)
    scales = np.asarray([1.0, 2.0 ** (1.0 / 3.0), 2.0 ** (2.0 / 3.0)],
                        dtype=np.float64)
    anchors = box_size * np.tile(scales, (2, len(ratios))).T
    areas = anchors[:, 0] * anchors[:, 1]
    anchors[:, 0] = np.sqrt(areas * np.repeat(ratios, len(scales)))
    anchors[:, 1] = anchors[:, 0] / np.repeat(ratios, len(scales))
    return anchors.astype(np.float32)


_C = 576
_N = 48960
_NDMA = 6
_CHUNK = _N // _NDMA

_LEVELS = (
    (64, 6, 8.0, 1, 64, 0, _anchor_table(32)),
    (32, 5, 16.0, 2, 16, 36864, _anchor_table(64)),
    (16, 4, 32.0, 4, 4, 46080, _anchor_table(128)),
    (8, 3, 64.0, 8, 1, 48384, _anchor_table(256)),
)


def _select9(a, consts):
    out = jnp.float32(float(consts[8]))
    for k in range(7, -1, -1):
        out = jnp.where(a == k, jnp.float32(float(consts[k])), out)
    return out


def _fill(buf_a, buf_x):
    i = jax.lax.broadcasted_iota(jnp.int32, (_C, 4), 0)
    j = jax.lax.broadcasted_iota(jnp.int32, (_C, 4), 1)
    for (W, log2w, s, hpc, reps, roff, tab) in _LEVELS:
        q = ((i.astype(jnp.float32) + 0.5) * (1.0 / 9.0)).astype(jnp.int32)
        a = i - q * 9
        w = (q & (W - 1)).astype(jnp.float32)
        h = (q >> log2w).astype(jnp.float32)
        cx = (w + 0.5) * s
        cy = (h + 0.5) * s
        wa = _select9(a, tab[:, 0])
        ha = _select9(a, tab[:, 1])
        cur_a = jnp.where(j == 0, cx,
                jnp.where(j == 1, cy,
                jnp.where(j == 2, wa, ha)))
        cur_x = jnp.where(j == 0, cx - 0.5 * wa,
                jnp.where(j == 1, cy - 0.5 * ha,
                jnp.where(j == 2, cx + 0.5 * wa, cy + 0.5 * ha)))
        step = jnp.float32(hpc * s)
        msk_a = jnp.where(j == 1, step, jnp.float32(0.0))
        msk_x = jnp.where((j & 1) == 1, step, jnp.float32(0.0))
        for g in range(reps):
            o = roff + g * _C
            buf_a[pl.ds(o, _C), :] = cur_a
            buf_x[pl.ds(o, _C), :] = cur_x
            if g + 1 < reps:
                cur_a = cur_a + msk_a
                cur_x = cur_x + msk_x


def _body(out_a_hbm, out_x_hbm, buf_a, buf_x, sem):
    _fill(buf_a, buf_x)
    cps = []
    for k in range(_NDMA):
        rows = pl.ds(k * _CHUNK, _CHUNK)
        cp_a = pltpu.make_async_copy(
            buf_a.at[rows, :], out_a_hbm.at[rows, :], sem.at[2 * k])
        cp_x = pltpu.make_async_copy(
            buf_x.at[rows, :], out_x_hbm.at[rows, :], sem.at[2 * k + 1])
        cp_a.start()
        cp_x.start()
        cps += [cp_a, cp_x]
    for cp in cps:
        cp.wait()


def kernel(feat0, feat1, feat2, feat3):
    del feat0, feat1, feat2, feat3
    return pl.pallas_call(
        _body,
        out_shape=[
            jax.ShapeDtypeStruct((_N, 4), jnp.float32),
            jax.ShapeDtypeStruct((_N, 4), jnp.float32),
        ],
        out_specs=[
            pl.BlockSpec(memory_space=pl.ANY),
            pl.BlockSpec(memory_space=pl.ANY),
        ],
        scratch_shapes=[
            pltpu.VMEM((_N, 4), jnp.float32),
            pltpu.VMEM((_N, 4), jnp.float32),
            pltpu.SemaphoreType.DMA((2 * _NDMA,)),
        ],
    )()
